# baseline (speedup 1.0000x reference)
import functools

import jax
import jax.numpy as jnp
from jax import lax
from jax.experimental import pallas as pl
from jax.experimental.pallas import tpu as pltpu
from jax.experimental.pallas import tpu_sc as plsc

N_NODES = 10000
N_EDGES = 160000
ND = 32
ED = 16
HD = 64

NUM_CORES = 2
NUM_SUBCORES = 16
NUM_WORKERS = NUM_CORES * NUM_SUBCORES
CHUNK = 40
NBUF = 5
HALF_A = 76800
HALF_B = N_EDGES - HALF_A
NPAD = 10112
ROWS_PER_SUBCORE = NPAD // NUM_SUBCORES

def _mesh():
    return plsc.VectorSubcoreMesh(core_axis_name="c", subcore_axis_name="s")


def _sc_gather(x, src_r, n_edges, nbuf):
    epw = n_edges // NUM_WORKERS
    nchunk = epw // CHUNK
    ngroup = nchunk // nbuf

    @functools.partial(
        pl.kernel,
        out_type=jax.ShapeDtypeStruct((n_edges, 128), jnp.float32),
        mesh=_mesh(),
        scratch_types=(
            [pltpu.VMEM((nchunk, CHUNK), jnp.int32)]
            + [pltpu.VMEM((CHUNK, 128), jnp.float32)] * nbuf
            + [pltpu.SemaphoreType.DMA] * (2 * nbuf)
        ),
    )
    def k(x_hbm, src_hbm, out_hbm, idx_v, *scr):
        bufs = scr[:nbuf]
        sg = scr[nbuf:2 * nbuf]
        so = scr[2 * nbuf:]
        wid = lax.axis_index("s") * NUM_CORES + lax.axis_index("c")
        pltpu.sync_copy(src_hbm.at[wid], idx_v)
        base = wid * epw

        def out_slice(j):
            return out_hbm.at[pl.ds(base + j * CHUNK, CHUNK)]

        for b in range(nbuf):
            pltpu.async_copy(x_hbm.at[idx_v.at[b]], bufs[b], sg[b])

        def body(g, carry):
            j0 = g * nbuf
            for b in range(nbuf):
                pltpu.make_async_copy(x_hbm.at[idx_v.at[j0 + b]],
                                      bufs[b], sg[b]).wait()
                pltpu.async_copy(bufs[b], out_slice(j0 + b), so[b])
            for b in range(nbuf):
                pltpu.make_async_copy(bufs[b], out_slice(j0 + b), so[b]).wait()

                @pl.when(g + 1 < ngroup)
                def _():
                    pltpu.async_copy(x_hbm.at[idx_v.at[j0 + nbuf + b]],
                                     bufs[b], sg[b])
            return carry

        lax.fori_loop(0, ngroup, body, 0)

    return k(x, src_r)


def _sc_scatter(messages, dst_r, zeros, n_edges, nbuf):
    epw = n_edges // NUM_WORKERS
    nchunk = epw // CHUNK
    ngroup = nchunk // nbuf

    @functools.partial(
        pl.kernel,
        out_type=jax.ShapeDtypeStruct((NUM_CORES, NPAD, 128), jnp.float32),
        mesh=_mesh(),
        scratch_types=(
            [pltpu.VMEM((nchunk, CHUNK), jnp.int32),
             pltpu.VMEM_SHARED((NPAD, 128), jnp.float32)]
            + [pltpu.VMEM((CHUNK, 128), jnp.float32)] * nbuf
            + [pltpu.SemaphoreType.DMA] * (2 * nbuf)
        ),
    )
    def k(msg_hbm, dst_hbm, zero_hbm, out_hbm, idx_v, agg_sh, *scr):
        bufs = scr[:nbuf]
        sr = scr[nbuf:2 * nbuf]
        sa = scr[2 * nbuf:]
        cid = lax.axis_index("c")
        sid = lax.axis_index("s")
        wid = sid * NUM_CORES + cid
        rows = pl.ds(sid * ROWS_PER_SUBCORE, ROWS_PER_SUBCORE)
        pltpu.sync_copy(zero_hbm, agg_sh.at[rows])
        pltpu.sync_copy(dst_hbm.at[wid], idx_v)
        plsc.subcore_barrier()
        base = wid * epw

        def msg_slice(j):
            return msg_hbm.at[pl.ds(base + j * CHUNK, CHUNK)]

        for b in range(nbuf):
            pltpu.async_copy(msg_slice(b), bufs[b], sr[b])

        def body(g, carry):
            j0 = g * nbuf
            for b in range(nbuf):
                pltpu.make_async_copy(msg_slice(j0 + b), bufs[b], sr[b]).wait()
                pltpu.async_copy(bufs[b], agg_sh.at[idx_v.at[j0 + b]],
                                 sa[b], add=True)
            for b in range(nbuf):
                pltpu.make_async_copy(bufs[b], agg_sh.at[idx_v.at[j0 + b]],
                                      sa[b]).wait()

                @pl.when(g + 1 < ngroup)
                def _():
                    pltpu.async_copy(msg_slice(j0 + nbuf + b), bufs[b], sr[b])
            return carry

        lax.fori_loop(0, ngroup, body, 0)
        plsc.subcore_barrier()
        pltpu.sync_copy(agg_sh.at[rows], out_hbm.at[cid, rows])

    return k(messages, dst_r, zeros)


def _tc_messages(edge_attr, x_src, W1T, b1r, W2T, bm, Rp, G, n_edges):
    ET = 3200
    GRID = n_edges // ET

    def body(ea_ref, xs_ref, w1_ref, b1_ref, w2_ref, bm_ref, rp_ref, g_ref,
             out_ref):
        h = jnp.dot(ea_ref[...], w1_ref[...],
                    preferred_element_type=jnp.float32) + b1_ref[...]
        h = 0.5 * h * (1.0 + lax.erf(h * 0.7071067811865476))
        xs = xs_ref[:, :ND]
        wt = jnp.dot(h.astype(jnp.bfloat16), w2_ref[...],
                     preferred_element_type=jnp.float32).astype(jnp.bfloat16)
        xsrep = jnp.dot(xs.astype(jnp.bfloat16), rp_ref[...],
                        preferred_element_type=jnp.float32).astype(jnp.bfloat16)
        y = wt * xsrep
        msg = (jnp.dot(y, g_ref[...], preferred_element_type=jnp.float32)
               + jnp.dot(xs, bm_ref[...], preferred_element_type=jnp.float32))
        out_ref[:, :ND] = msg

    return pl.pallas_call(
        body,
        grid=(GRID,),
        in_specs=[
            pl.BlockSpec((ET, ED), lambda i: (i, 0)),
            pl.BlockSpec((ET, 128), lambda i: (i, 0)),
            pl.BlockSpec((ED, HD), lambda i: (0, 0)),
            pl.BlockSpec((1, HD), lambda i: (0, 0)),
            pl.BlockSpec((HD, ND * ND), lambda i: (0, 0)),
            pl.BlockSpec((ND, ND), lambda i: (0, 0)),
            pl.BlockSpec((ND, ND * ND), lambda i: (0, 0)),
            pl.BlockSpec((ND * ND, ND), lambda i: (0, 0)),
        ],
        out_specs=pl.BlockSpec((ET, 128), lambda i: (i, 0)),
        out_shape=jax.ShapeDtypeStruct((n_edges, 128), jnp.float32),
    )(edge_attr, x_src, W1T, b1r, W2T, bm, Rp, G)


def _tc_gru(x, parts_a, parts_b, W_ihT, b_ihr, W_hhT, b_hhr):
    def body(x_ref, a_ref, b_ref, wih_ref, bih_ref, whh_ref, bhh_ref,
             out_ref):
        agg = (a_ref[0, :N_NODES, :ND] + a_ref[1, :N_NODES, :ND]
               + b_ref[0, :N_NODES, :ND] + b_ref[1, :N_NODES, :ND])
        gi = jnp.dot(agg, wih_ref[...],
                     preferred_element_type=jnp.float32) + bih_ref[...]
        gh = jnp.dot(x_ref[...], whh_ref[...],
                     preferred_element_type=jnp.float32) + bhh_ref[...]
        r = jax.nn.sigmoid(gi[:, :ND] + gh[:, :ND])
        z = jax.nn.sigmoid(gi[:, ND:2 * ND] + gh[:, ND:2 * ND])
        n = jnp.tanh(gi[:, 2 * ND:] + r * gh[:, 2 * ND:])
        out_ref[...] = (1.0 - z) * n + z * x_ref[...]

    return pl.pallas_call(
        body,
        out_shape=jax.ShapeDtypeStruct((N_NODES, ND), jnp.float32),
    )(x, parts_a, parts_b, W_ihT, b_ihr, W_hhT, b_hhr)


def kernel(x, edge_index, edge_attr, W1, b1, W2, b2, W_ih, W_hh, b_ih, b_hh):
    src = edge_index[0]
    dst = edge_index[1]
    src_a = src[:HALF_A].reshape(NUM_WORKERS, -1, CHUNK)
    src_b = src[HALF_A:].reshape(NUM_WORKERS, -1, CHUNK)
    dst_a = dst[:HALF_A].reshape(NUM_WORKERS, -1, CHUNK)
    dst_b = dst[HALF_A:].reshape(NUM_WORKERS, -1, CHUNK)
    ea_a = edge_attr[:HALF_A]
    ea_b = edge_attr[HALF_A:]
    Rp = jnp.kron(jnp.ones((1, ND), dtype=jnp.bfloat16),
                  jnp.eye(ND, dtype=jnp.bfloat16))
    G = jnp.kron(jnp.eye(ND, dtype=jnp.bfloat16),
                 jnp.ones((ND, 1), dtype=jnp.bfloat16))
    Bm = b2.reshape(ND, ND).T
    zeros = jnp.zeros((ROWS_PER_SUBCORE, 128), dtype=jnp.float32)
    W1T = W1.T
    b1r = b1.reshape(1, HD)
    W2Tb = W2.T.astype(jnp.bfloat16)

    x128 = jnp.pad(x, ((0, 0), (0, 128 - ND)))
    xa = _sc_gather(x128, src_a, HALF_A, 12)
    xb = _sc_gather(x128, src_b, HALF_B, 13)
    ma = _tc_messages(ea_a, xa, W1T, b1r, W2Tb, Bm, Rp, G, HALF_A)
    pa = _sc_scatter(ma, dst_a, zeros, HALF_A, 5)
    mb = _tc_messages(ea_b, xb, W1T, b1r, W2Tb, Bm, Rp, G, HALF_B)
    pb = _sc_scatter(mb, dst_b, zeros, HALF_B, 5)
    return _tc_gru(x, pa, pb, W_ih.T, b_ih.reshape(1, 3 * ND),
                   W_hh.T, b_hh.reshape(1, 3 * ND))


# speedup vs baseline: 1.0035x; 1.0035x over previous
import functools

import jax
import jax.numpy as jnp
from jax import lax
from jax.experimental import pallas as pl
from jax.experimental.pallas import tpu as pltpu
from jax.experimental.pallas import tpu_sc as plsc

N_NODES = 10000
N_EDGES = 160000
ND = 32
ED = 16
HD = 64

NUM_CORES = 2
NUM_SUBCORES = 16
NUM_WORKERS = NUM_CORES * NUM_SUBCORES
CHUNK = 40
NBUF = 5
HALF_A = 76800
HALF_B = N_EDGES - HALF_A
NPAD = 10112
ROWS_PER_SUBCORE = NPAD // NUM_SUBCORES

def _mesh():
    return plsc.VectorSubcoreMesh(core_axis_name="c", subcore_axis_name="s")


def _sc_gather(x, src_r, n_edges, nbuf):
    epw = n_edges // NUM_WORKERS
    nchunk = epw // CHUNK
    ngroup = nchunk // nbuf

    @functools.partial(
        pl.kernel,
        out_type=jax.ShapeDtypeStruct((n_edges, 128), jnp.float32),
        mesh=_mesh(),
        scratch_types=(
            [pltpu.VMEM((nchunk, CHUNK), jnp.int32)]
            + [pltpu.VMEM((CHUNK, 128), jnp.float32)] * nbuf
            + [pltpu.SemaphoreType.DMA] * (2 * nbuf)
        ),
    )
    def k(x_hbm, src_hbm, out_hbm, idx_v, *scr):
        bufs = scr[:nbuf]
        sg = scr[nbuf:2 * nbuf]
        so = scr[2 * nbuf:]
        wid = lax.axis_index("s") * NUM_CORES + lax.axis_index("c")
        pltpu.sync_copy(src_hbm.at[wid], idx_v)
        base = wid * epw

        def out_slice(j):
            return out_hbm.at[pl.ds(base + j * CHUNK, CHUNK)]

        for b in range(nbuf):
            pltpu.async_copy(x_hbm.at[idx_v.at[b]], bufs[b], sg[b])

        def body(g, carry):
            j0 = g * nbuf
            for b in range(nbuf):
                pltpu.make_async_copy(x_hbm.at[idx_v.at[j0 + b]],
                                      bufs[b], sg[b]).wait()
                pltpu.async_copy(bufs[b], out_slice(j0 + b), so[b])
            for b in range(nbuf):
                pltpu.make_async_copy(bufs[b], out_slice(j0 + b), so[b]).wait()

                @pl.when(g + 1 < ngroup)
                def _():
                    pltpu.async_copy(x_hbm.at[idx_v.at[j0 + nbuf + b]],
                                     bufs[b], sg[b])
            return carry

        lax.fori_loop(0, ngroup, body, 0)

    return k(x, src_r)


def _sc_scatter(messages, dst_r, zeros, n_edges, nbuf):
    epw = n_edges // NUM_WORKERS
    nchunk = epw // CHUNK
    ngroup = nchunk // nbuf

    @functools.partial(
        pl.kernel,
        out_type=jax.ShapeDtypeStruct((NUM_CORES, NPAD, 128), jnp.float32),
        mesh=_mesh(),
        scratch_types=(
            [pltpu.VMEM((nchunk, CHUNK), jnp.int32),
             pltpu.VMEM_SHARED((NPAD, 128), jnp.float32)]
            + [pltpu.VMEM((CHUNK, 128), jnp.float32)] * nbuf
            + [pltpu.SemaphoreType.DMA] * (2 * nbuf)
        ),
    )
    def k(msg_hbm, dst_hbm, zero_hbm, out_hbm, idx_v, agg_sh, *scr):
        bufs = scr[:nbuf]
        sr = scr[nbuf:2 * nbuf]
        sa = scr[2 * nbuf:]
        cid = lax.axis_index("c")
        sid = lax.axis_index("s")
        wid = sid * NUM_CORES + cid
        rows = pl.ds(sid * ROWS_PER_SUBCORE, ROWS_PER_SUBCORE)
        pltpu.sync_copy(zero_hbm, agg_sh.at[rows])
        pltpu.sync_copy(dst_hbm.at[wid], idx_v)
        plsc.subcore_barrier()
        base = wid * epw

        def msg_slice(j):
            return msg_hbm.at[pl.ds(base + j * CHUNK, CHUNK)]

        for b in range(nbuf):
            pltpu.async_copy(msg_slice(b), bufs[b], sr[b])

        def body(g, carry):
            j0 = g * nbuf
            for b in range(nbuf):
                pltpu.make_async_copy(msg_slice(j0 + b), bufs[b], sr[b]).wait()
                pltpu.async_copy(bufs[b], agg_sh.at[idx_v.at[j0 + b]],
                                 sa[b], add=True)
            for b in range(nbuf):
                pltpu.make_async_copy(bufs[b], agg_sh.at[idx_v.at[j0 + b]],
                                      sa[b]).wait()

                @pl.when(g + 1 < ngroup)
                def _():
                    pltpu.async_copy(msg_slice(j0 + nbuf + b), bufs[b], sr[b])
            return carry

        lax.fori_loop(0, ngroup, body, 0)
        plsc.subcore_barrier()
        pltpu.sync_copy(agg_sh.at[rows], out_hbm.at[cid, rows])

    return k(messages, dst_r, zeros)


def _tc_messages(edge_attr, x_src, W1T, b1r, W2T, bm, Rp, G, n_edges):
    ET = 3200
    GRID = n_edges // ET

    def body(ea_ref, xs_ref, w1_ref, b1_ref, w2_ref, bm_ref, rp_ref, g_ref,
             out_ref):
        h = jnp.dot(ea_ref[...], w1_ref[...],
                    preferred_element_type=jnp.float32) + b1_ref[...]
        h = 0.5 * h * (1.0 + lax.erf(h * 0.7071067811865476))
        xs = xs_ref[:, :ND]
        wt = jnp.dot(h.astype(jnp.bfloat16), w2_ref[...],
                     preferred_element_type=jnp.float32).astype(jnp.bfloat16)
        xsrep = jnp.dot(xs.astype(jnp.bfloat16), rp_ref[...],
                        preferred_element_type=jnp.float32).astype(jnp.bfloat16)
        y = wt * xsrep
        msg = (jnp.dot(y, g_ref[...], preferred_element_type=jnp.float32)
               + jnp.dot(xs, bm_ref[...], preferred_element_type=jnp.float32))
        out_ref[:, :ND] = msg

    return pl.pallas_call(
        body,
        grid=(GRID,),
        in_specs=[
            pl.BlockSpec((ET, ED), lambda i: (i, 0)),
            pl.BlockSpec((ET, 128), lambda i: (i, 0)),
            pl.BlockSpec((ED, HD), lambda i: (0, 0)),
            pl.BlockSpec((1, HD), lambda i: (0, 0)),
            pl.BlockSpec((HD, ND * ND), lambda i: (0, 0)),
            pl.BlockSpec((ND, ND), lambda i: (0, 0)),
            pl.BlockSpec((ND, ND * ND), lambda i: (0, 0)),
            pl.BlockSpec((ND * ND, ND), lambda i: (0, 0)),
        ],
        out_specs=pl.BlockSpec((ET, 128), lambda i: (i, 0)),
        out_shape=jax.ShapeDtypeStruct((n_edges, 128), jnp.float32),
    )(edge_attr, x_src, W1T, b1r, W2T, bm, Rp, G)


def _tc_gru(x, parts_a, parts_b, W_ihT, b_ihr, W_hhT, b_hhr):
    def body(x_ref, a_ref, b_ref, wih_ref, bih_ref, whh_ref, bhh_ref,
             out_ref):
        agg = (a_ref[0, :N_NODES, :ND] + a_ref[1, :N_NODES, :ND]
               + b_ref[0, :N_NODES, :ND] + b_ref[1, :N_NODES, :ND])
        gi = jnp.dot(agg, wih_ref[...],
                     preferred_element_type=jnp.float32) + bih_ref[...]
        gh = jnp.dot(x_ref[...], whh_ref[...],
                     preferred_element_type=jnp.float32) + bhh_ref[...]
        r = jax.nn.sigmoid(gi[:, :ND] + gh[:, :ND])
        z = jax.nn.sigmoid(gi[:, ND:2 * ND] + gh[:, ND:2 * ND])
        n = jnp.tanh(gi[:, 2 * ND:] + r * gh[:, 2 * ND:])
        out_ref[...] = (1.0 - z) * n + z * x_ref[...]

    return pl.pallas_call(
        body,
        out_shape=jax.ShapeDtypeStruct((N_NODES, ND), jnp.float32),
    )(x, parts_a, parts_b, W_ihT, b_ihr, W_hhT, b_hhr)


def kernel(x, edge_index, edge_attr, W1, b1, W2, b2, W_ih, W_hh, b_ih, b_hh):
    src = edge_index[0]
    dst = edge_index[1]
    src_a = src[:HALF_A].reshape(NUM_WORKERS, -1, CHUNK)
    src_b = src[HALF_A:].reshape(NUM_WORKERS, -1, CHUNK)
    dst_a = dst[:HALF_A].reshape(NUM_WORKERS, -1, CHUNK)
    dst_b = dst[HALF_A:].reshape(NUM_WORKERS, -1, CHUNK)
    ea_a = edge_attr[:HALF_A]
    ea_b = edge_attr[HALF_A:]
    Rp = jnp.kron(jnp.ones((1, ND), dtype=jnp.bfloat16),
                  jnp.eye(ND, dtype=jnp.bfloat16))
    G = jnp.kron(jnp.eye(ND, dtype=jnp.bfloat16),
                 jnp.ones((ND, 1), dtype=jnp.bfloat16))
    Bm = b2.reshape(ND, ND).T
    zeros = jnp.zeros((ROWS_PER_SUBCORE, 128), dtype=jnp.float32)
    W1T = W1.T
    b1r = b1.reshape(1, HD)
    W2Tb = W2.T.astype(jnp.bfloat16)

    x128 = jnp.pad(x, ((0, 0), (0, 128 - ND)))
    xa = _sc_gather(x128, src_a, HALF_A, NBUF)
    xb = _sc_gather(x128, src_b, HALF_B, NBUF)
    ma = _tc_messages(ea_a, xa, W1T, b1r, W2Tb, Bm, Rp, G, HALF_A)
    pa = _sc_scatter(ma, dst_a, zeros, HALF_A, NBUF)
    mb = _tc_messages(ea_b, xb, W1T, b1r, W2Tb, Bm, Rp, G, HALF_B)
    pb = _sc_scatter(mb, dst_b, zeros, HALF_B, NBUF)
    return _tc_gru(x, pa, pb, W_ih.T, b_ih.reshape(1, 3 * ND),
                   W_hh.T, b_hh.reshape(1, 3 * ND))
